# baseline (speedup 1.0000x reference)
import functools

import jax
import jax.numpy as jnp
from jax import lax
from jax.experimental import pallas as pl
from jax.experimental.pallas import tpu as pltpu
from jax.experimental.pallas import tpu_sc as plsc

_NUM_CORES = 2
_NUM_SUBCORES = 16
_NUM_WORKERS = _NUM_CORES * _NUM_SUBCORES
_CHUNK_ROWS = 32


def kernel(x, pos_table):
    batch, seq_len = x.shape
    d_model = pos_table.shape[1]
    pos = pos_table[:seq_len]
    rows_per_w = seq_len // _NUM_WORKERS
    n_chunks = rows_per_w // _CHUNK_ROWS
    mesh = plsc.VectorSubcoreMesh(
        core_axis_name="c", subcore_axis_name="s",
        num_cores=_NUM_CORES, num_subcores=_NUM_SUBCORES)

    @functools.partial(
        pl.kernel,
        out_type=jax.ShapeDtypeStruct((batch * seq_len, d_model), pos_table.dtype),
        mesh=mesh,
        scratch_types=[
            pltpu.VMEM((_CHUNK_ROWS, d_model), jnp.float32),
        ],
    )
    def copy_kernel(pos_hbm, out_hbm, buf):
        wid = lax.axis_index("s") * _NUM_CORES + lax.axis_index("c")
        base = wid * rows_per_w

        def body(c, carry):
            r = base + c * _CHUNK_ROWS
            pltpu.sync_copy(pos_hbm.at[pl.ds(r, _CHUNK_ROWS)], buf)
            for b in range(batch):
                pltpu.sync_copy(buf, out_hbm.at[pl.ds(b * seq_len + r, _CHUNK_ROWS)])
            return carry

        lax.fori_loop(0, n_chunks, body, 0)

    flat = copy_kernel(pos)
    return flat.reshape(batch, seq_len, d_model)
